# baseline (speedup 1.0000x reference)
import functools
import math

import jax
import jax.numpy as jnp
from jax import lax
from jax.experimental import pallas as pl
from jax.experimental.pallas import tpu as pltpu
from jax.experimental.pallas import tpu_sc as plsc

ACTIVE_THRESHOLD = 0.01
NUM_CORES = 2
NUM_SUBCORES = 16
NUM_WORKERS = NUM_CORES * NUM_SUBCORES
LANES = 16
CHUNK = 1024
STREAM_ROWS = 128


def _sc_accumulate(nnz_pad, rows, cols_total, batch, interpret=False):
  per_worker = nnz_pad // NUM_WORKERS
  n_sub = per_worker // CHUNK
  assert n_sub % 2 == 0
  groups = CHUNK // LANES
  n_streams = CHUNK // STREAM_ROWS
  rows_per_sub = rows // NUM_SUBCORES
  mask_iters = batch * cols_total // LANES

  mesh = plsc.VectorSubcoreMesh(
      core_axis_name="c", subcore_axis_name="s", num_cores=NUM_CORES,
      num_subcores=NUM_SUBCORES)

  @functools.partial(
      pl.kernel,
      out_type=jax.ShapeDtypeStruct((NUM_CORES, rows, LANES), jnp.float32),
      mesh=mesh,
      scratch_types=[
          pltpu.VMEM((batch * cols_total,), jnp.float32),
          pltpu.VMEM((2, CHUNK), jnp.int32),
          pltpu.VMEM((2, CHUNK), jnp.float32),
          pltpu.VMEM((2, n_streams, STREAM_ROWS), jnp.int32),
          pltpu.VMEM((2, CHUNK, LANES), jnp.float32),
          pltpu.VMEM_SHARED((rows, LANES), jnp.float32),
          pltpu.SemaphoreType.DMA,
          pltpu.SemaphoreType.DMA,
          pltpu.SemaphoreType.DMA,
          pltpu.SemaphoreType.DMA,
      ],
      compiler_params=pltpu.CompilerParams(
          needs_layout_passes=False, use_tc_tiling_on_sc=False),
      interpret=interpret,
  )
  def run(x_hbm, cols_hbm, vals_hbm, rows_hbm, zeros_hbm, out_hbm,
          xv, cbuf, vbuf, rbuf, stg, acc, in0, in1, st0, st1):
    cid = lax.axis_index("c")
    sid = lax.axis_index("s")
    wid = cid * NUM_SUBCORES + sid
    in_sems = (in0, in1)
    st_sems = (st0, st1)

    pltpu.sync_copy(x_hbm, xv)

    pltpu.sync_copy(zeros_hbm, acc.at[pl.ds(sid * rows_per_sub, rows_per_sub)])

    zero16 = jnp.zeros((LANES,), jnp.float32)

    @pl.loop(0, mask_iters)
    def _mask(i):
      v = xv[pl.ds(i * LANES, LANES)]
      xv[pl.ds(i * LANES, LANES)] = jnp.where(v > ACTIVE_THRESHOLD, v, zero16)

    plsc.subcore_barrier()

    iota16 = lax.iota(jnp.int32, LANES)
    base = wid * per_worker

    def fire_in(j, slot):
      off = base + j * CHUNK
      roff = pl.multiple_of(off // STREAM_ROWS, 8)
      pltpu.async_copy(cols_hbm.at[pl.ds(off, CHUNK)], cbuf.at[slot],
                       in_sems[slot])
      pltpu.async_copy(vals_hbm.at[pl.ds(off, CHUNK)], vbuf.at[slot],
                       in_sems[slot])
      pltpu.async_copy(rows_hbm.at[pl.ds(roff, n_streams)], rbuf.at[slot],
                       in_sems[slot])

    def wait_in(slot):
      pltpu.make_async_copy(cols_hbm.at[pl.ds(0, CHUNK)], cbuf.at[slot],
                            in_sems[slot]).wait()
      pltpu.make_async_copy(vals_hbm.at[pl.ds(0, CHUNK)], vbuf.at[slot],
                            in_sems[slot]).wait()
      pltpu.make_async_copy(rows_hbm.at[pl.ds(0, n_streams)], rbuf.at[slot],
                            in_sems[slot]).wait()

    def compute(slot):
      @plsc.parallel_loop(0, groups, unroll=2)
      def _group(g):
        colv = cbuf[slot, pl.ds(g * LANES, LANES)]
        valv = vbuf[slot, pl.ds(g * LANES, LANES)]
        ridx = g * LANES + iota16
        contribs = []
        for b in range(batch):
          gathered = plsc.load_gather(xv, [colv + b * cols_total])
          contribs.append(gathered * valv)
        for b in range(batch):
          plsc.store_scatter(stg.at[slot],
                             [ridx, jnp.full((LANES,), b, jnp.int32)],
                             contribs[b])

    def fire_streams(slot):
      for i in range(n_streams):
        pltpu.async_copy(
            stg.at[slot].at[pl.ds(i * STREAM_ROWS, STREAM_ROWS)],
            acc.at[rbuf.at[slot].at[i]], st_sems[slot], add=True)

    def drain_streams(slot):
      for i in range(n_streams):
        pltpu.make_async_copy(
            stg.at[slot].at[pl.ds(i * STREAM_ROWS, STREAM_ROWS)],
            acc.at[rbuf.at[slot].at[i]], st_sems[slot]).wait()

    fire_in(0, 0)

    @pl.loop(0, n_sub // 2)
    def _pipeline(jj):
      j0 = jj * 2
      wait_in(0)
      compute(0)
      fire_streams(0)

      @pl.when(jj > 0)
      def _drain1():
        drain_streams(1)

      fire_in(j0 + 1, 1)
      wait_in(1)
      compute(1)
      fire_streams(1)
      drain_streams(0)

      @pl.when(j0 + 2 < n_sub)
      def _prefetch():
        fire_in(j0 + 2, 0)

    drain_streams(1)

    plsc.subcore_barrier()

    r0 = sid * rows_per_sub
    pltpu.sync_copy(acc.at[pl.ds(r0, rows_per_sub)],
                    out_hbm.at[cid, pl.ds(r0, rows_per_sub)])

  return run


def _combine_kernel(p_ref, b_ref, o_ref):
  s = p_ref[0] + p_ref[1]
  o_ref[...] = s.T + b_ref[...]


@jax.jit
def kernel(x, row_indices, col_indices, values, bias):
  batch, cols_total = x.shape
  rows = bias.shape[0]
  nnz = row_indices.shape[0]
  nnz_pad = math.ceil(nnz / (NUM_WORKERS * 2 * CHUNK)) * NUM_WORKERS * 2 * CHUNK
  pad = nnz_pad - nnz

  cols_p = jnp.concatenate([col_indices, jnp.zeros((pad,), jnp.int32)])
  vals_p = jnp.concatenate([values, jnp.zeros((pad,), jnp.float32)])
  rows_p = jnp.concatenate([row_indices, jnp.zeros((pad,), jnp.int32)])
  rows_2d = rows_p.reshape(nnz_pad // STREAM_ROWS, STREAM_ROWS)
  zeros_tile = jnp.zeros((rows // NUM_SUBCORES, LANES), jnp.float32)

  partials = _sc_accumulate(nnz_pad, rows, cols_total, batch)(
      x.reshape(-1), cols_p, vals_p, rows_2d, zeros_tile)

  out = pl.pallas_call(
      _combine_kernel,
      out_shape=jax.ShapeDtypeStruct((batch, rows), jnp.float32),
  )(partials, bias.reshape(1, rows))
  return out


# speedup vs baseline: 1.6027x; 1.6027x over previous
import functools
import math

import jax
import jax.numpy as jnp
from jax import lax
from jax.experimental import pallas as pl
from jax.experimental.pallas import tpu as pltpu
from jax.experimental.pallas import tpu_sc as plsc

ACTIVE_THRESHOLD = 0.01
NUM_CORES = 2
NUM_SUBCORES = 16
NUM_WORKERS = NUM_CORES * NUM_SUBCORES
LANES = 16
CHUNK = 1024
STREAM_ROWS = 128


def _sc_accumulate(nnz_pad, rows, cols_total, batch, interpret=False):
  per_worker = nnz_pad // NUM_WORKERS
  n_sub = per_worker // CHUNK
  assert n_sub % 2 == 0
  groups = CHUNK // LANES
  n_streams = CHUNK // STREAM_ROWS
  rows_per_sub = rows // NUM_SUBCORES
  mask_iters = batch * cols_total // LANES

  mesh = plsc.VectorSubcoreMesh(
      core_axis_name="c", subcore_axis_name="s", num_cores=NUM_CORES,
      num_subcores=NUM_SUBCORES)

  @functools.partial(
      pl.kernel,
      out_type=jax.ShapeDtypeStruct((NUM_CORES, rows, LANES), jnp.float32),
      mesh=mesh,
      scratch_types=[
          pltpu.VMEM((batch * cols_total,), jnp.float32),
          pltpu.VMEM((2, CHUNK), jnp.int32),
          pltpu.VMEM((2, CHUNK), jnp.float32),
          pltpu.VMEM((2, n_streams, STREAM_ROWS), jnp.int32),
          pltpu.VMEM((2, CHUNK, LANES), jnp.float32),
          pltpu.VMEM_SHARED((rows, LANES), jnp.float32),
          pltpu.SemaphoreType.DMA,
          pltpu.SemaphoreType.DMA,
          pltpu.SemaphoreType.DMA,
          pltpu.SemaphoreType.DMA,
      ],
      compiler_params=pltpu.CompilerParams(
          needs_layout_passes=False, use_tc_tiling_on_sc=False),
      interpret=interpret,
  )
  def run(x_hbm, cols_hbm, vals_hbm, rows_hbm, zeros_hbm, out_hbm,
          xv, cbuf, vbuf, rbuf, stg, acc, in0, in1, st0, st1):
    cid = lax.axis_index("c")
    sid = lax.axis_index("s")
    wid = cid * NUM_SUBCORES + sid
    in_sems = (in0, in1)
    st_sems = (st0, st1)

    pltpu.sync_copy(x_hbm, xv)

    pltpu.sync_copy(zeros_hbm, acc.at[pl.ds(sid * rows_per_sub, rows_per_sub)])

    zero16 = jnp.zeros((LANES,), jnp.float32)

    @pl.loop(0, mask_iters)
    def _mask(i):
      v = xv[pl.ds(i * LANES, LANES)]
      xv[pl.ds(i * LANES, LANES)] = jnp.where(v > ACTIVE_THRESHOLD, v, zero16)

    plsc.subcore_barrier()

    iota16 = lax.iota(jnp.int32, LANES)
    base = wid * per_worker

    def fire_in(j, slot):
      off = base + j * CHUNK
      roff = pl.multiple_of(off // STREAM_ROWS, 8)
      pltpu.async_copy(cols_hbm.at[pl.ds(off, CHUNK)], cbuf.at[slot],
                       in_sems[slot])
      pltpu.async_copy(vals_hbm.at[pl.ds(off, CHUNK)], vbuf.at[slot],
                       in_sems[slot])
      pltpu.async_copy(rows_hbm.at[pl.ds(roff, n_streams)], rbuf.at[slot],
                       in_sems[slot])

    def wait_in(slot):
      pltpu.make_async_copy(cols_hbm.at[pl.ds(0, CHUNK)], cbuf.at[slot],
                            in_sems[slot]).wait()
      pltpu.make_async_copy(vals_hbm.at[pl.ds(0, CHUNK)], vbuf.at[slot],
                            in_sems[slot]).wait()
      pltpu.make_async_copy(rows_hbm.at[pl.ds(0, n_streams)], rbuf.at[slot],
                            in_sems[slot]).wait()

    def compute(slot):
      @plsc.parallel_loop(0, groups)
      def _group(g):
        colv = cbuf[slot, pl.ds(g * LANES, LANES)]
        valv = vbuf[slot, pl.ds(g * LANES, LANES)]
        ridx = g * LANES + iota16
        contribs = []
        for b in range(batch):
          gathered = plsc.load_gather(xv, [colv + b * cols_total])
          contribs.append(gathered * valv)
        for b in range(batch):
          plsc.store_scatter(stg.at[slot],
                             [ridx, jnp.full((LANES,), b, jnp.int32)],
                             contribs[b])

    def fire_streams(slot):
      for i in range(n_streams):
        pltpu.async_copy(
            stg.at[slot].at[pl.ds(i * STREAM_ROWS, STREAM_ROWS)],
            acc.at[rbuf.at[slot].at[i]], st_sems[slot], add=True)

    def drain_streams(slot):
      for i in range(n_streams):
        pltpu.make_async_copy(
            stg.at[slot].at[pl.ds(i * STREAM_ROWS, STREAM_ROWS)],
            acc.at[rbuf.at[slot].at[i]], st_sems[slot]).wait()

    fire_in(0, 0)

    @pl.loop(0, n_sub // 2)
    def _pipeline(jj):
      j0 = jj * 2
      wait_in(0)
      compute(0)
      fire_streams(0)

      @pl.when(jj > 0)
      def _drain1():
        drain_streams(1)

      fire_in(j0 + 1, 1)
      wait_in(1)
      compute(1)
      fire_streams(1)
      drain_streams(0)

      @pl.when(j0 + 2 < n_sub)
      def _prefetch():
        fire_in(j0 + 2, 0)

    drain_streams(1)

    plsc.subcore_barrier()

    r0 = sid * rows_per_sub
    pltpu.sync_copy(acc.at[pl.ds(r0, rows_per_sub)],
                    out_hbm.at[cid, pl.ds(r0, rows_per_sub)])

  return run


def _combine_kernel(p_ref, b_ref, o_ref):
  s = p_ref[0] + p_ref[1]
  o_ref[...] = s.T + b_ref[...]


@jax.jit
def kernel(x, row_indices, col_indices, values, bias):
  batch, cols_total = x.shape
  rows = bias.shape[0]
  nnz = row_indices.shape[0]
  nnz_pad = math.ceil(nnz / (NUM_WORKERS * 2 * CHUNK)) * NUM_WORKERS * 2 * CHUNK
  pad = nnz_pad - nnz

  cols_p = jnp.concatenate([col_indices, jnp.zeros((pad,), jnp.int32)])
  vals_p = jnp.concatenate([values, jnp.zeros((pad,), jnp.float32)])
  rows_p = jnp.concatenate([row_indices, jnp.zeros((pad,), jnp.int32)])
  rows_2d = rows_p.reshape(nnz_pad // STREAM_ROWS, STREAM_ROWS)
  zeros_tile = jnp.zeros((rows // NUM_SUBCORES, LANES), jnp.float32)

  partials = _sc_accumulate(nnz_pad, rows, cols_total, batch)(
      x.reshape(-1), cols_p, vals_p, rows_2d, zeros_tile)

  out = pl.pallas_call(
      _combine_kernel,
      out_shape=jax.ShapeDtypeStruct((batch, rows), jnp.float32),
  )(partials, bias.reshape(1, rows))
  return out


# speedup vs baseline: 1.6583x; 1.0347x over previous
import functools
import math

import jax
import jax.numpy as jnp
from jax import lax
from jax.experimental import pallas as pl
from jax.experimental.pallas import tpu as pltpu
from jax.experimental.pallas import tpu_sc as plsc

ACTIVE_THRESHOLD = 0.01
NUM_CORES = 2
NUM_SUBCORES = 16
NUM_WORKERS = NUM_CORES * NUM_SUBCORES
LANES = 16
CHUNK = 1024
STREAM_ROWS = 128


def _sc_accumulate(nnz_pad, rows, cols_total, batch, interpret=False):
  per_worker = nnz_pad // NUM_WORKERS
  n_sub = per_worker // CHUNK
  assert n_sub % 2 == 0
  groups = CHUNK // LANES
  n_streams = CHUNK // STREAM_ROWS
  rows_per_sub = rows // NUM_SUBCORES
  mask_iters = batch * cols_total // LANES

  mesh = plsc.VectorSubcoreMesh(
      core_axis_name="c", subcore_axis_name="s", num_cores=NUM_CORES,
      num_subcores=NUM_SUBCORES)

  @functools.partial(
      pl.kernel,
      out_type=jax.ShapeDtypeStruct((NUM_CORES, rows, LANES), jnp.float32),
      mesh=mesh,
      scratch_types=[
          pltpu.VMEM((batch * cols_total,), jnp.float32),
          pltpu.VMEM((2, CHUNK), jnp.int32),
          pltpu.VMEM((2, CHUNK), jnp.float32),
          pltpu.VMEM((2, n_streams, STREAM_ROWS), jnp.int32),
          pltpu.VMEM((2, CHUNK, LANES), jnp.float32),
          pltpu.VMEM_SHARED((rows, LANES), jnp.float32),
          pltpu.SemaphoreType.DMA,
          pltpu.SemaphoreType.DMA,
          pltpu.SemaphoreType.DMA,
          pltpu.SemaphoreType.DMA,
      ],
      compiler_params=pltpu.CompilerParams(
          needs_layout_passes=False, use_tc_tiling_on_sc=False),
      interpret=interpret,
  )
  def run(x_hbm, cols_hbm, vals_hbm, rows_hbm, zeros_hbm, out_hbm,
          xv, cbuf, vbuf, rbuf, stg, acc, in0, in1, st0, st1):
    cid = lax.axis_index("c")
    sid = lax.axis_index("s")
    wid = cid * NUM_SUBCORES + sid
    in_sems = (in0, in1)
    st_sems = (st0, st1)

    pltpu.sync_copy(x_hbm, xv)

    pltpu.sync_copy(zeros_hbm, acc.at[pl.ds(sid * rows_per_sub, rows_per_sub)])

    zero16 = jnp.zeros((LANES,), jnp.float32)

    @plsc.parallel_loop(0, mask_iters)
    def _mask(i):
      v = xv[pl.ds(i * LANES, LANES)]
      xv[pl.ds(i * LANES, LANES)] = jnp.where(v > ACTIVE_THRESHOLD, v, zero16)

    plsc.subcore_barrier()

    iota16 = lax.iota(jnp.int32, LANES)
    base = wid * per_worker

    def fire_in(j, slot):
      off = base + j * CHUNK
      roff = pl.multiple_of(off // STREAM_ROWS, 8)
      pltpu.async_copy(cols_hbm.at[pl.ds(off, CHUNK)], cbuf.at[slot],
                       in_sems[slot])
      pltpu.async_copy(vals_hbm.at[pl.ds(off, CHUNK)], vbuf.at[slot],
                       in_sems[slot])
      pltpu.async_copy(rows_hbm.at[pl.ds(roff, n_streams)], rbuf.at[slot],
                       in_sems[slot])

    def wait_in(slot):
      pltpu.make_async_copy(cols_hbm.at[pl.ds(0, CHUNK)], cbuf.at[slot],
                            in_sems[slot]).wait()
      pltpu.make_async_copy(vals_hbm.at[pl.ds(0, CHUNK)], vbuf.at[slot],
                            in_sems[slot]).wait()
      pltpu.make_async_copy(rows_hbm.at[pl.ds(0, n_streams)], rbuf.at[slot],
                            in_sems[slot]).wait()

    def compute(slot):
      @plsc.parallel_loop(0, groups)
      def _group(g):
        colv = cbuf[slot, pl.ds(g * LANES, LANES)]
        valv = vbuf[slot, pl.ds(g * LANES, LANES)]
        ridx = g * LANES + iota16
        contribs = []
        for b in range(batch):
          gathered = plsc.load_gather(xv, [colv + b * cols_total])
          contribs.append(gathered * valv)
        for b in range(batch):
          plsc.store_scatter(stg.at[slot],
                             [ridx, jnp.full((LANES,), b, jnp.int32)],
                             contribs[b])

    def fire_streams(slot):
      for i in range(n_streams):
        pltpu.async_copy(
            stg.at[slot].at[pl.ds(i * STREAM_ROWS, STREAM_ROWS)],
            acc.at[rbuf.at[slot].at[i]], st_sems[slot], add=True)

    def drain_streams(slot):
      for i in range(n_streams):
        pltpu.make_async_copy(
            stg.at[slot].at[pl.ds(i * STREAM_ROWS, STREAM_ROWS)],
            acc.at[rbuf.at[slot].at[i]], st_sems[slot]).wait()

    fire_in(0, 0)

    @pl.loop(0, n_sub // 2)
    def _pipeline(jj):
      j0 = jj * 2
      wait_in(0)
      compute(0)
      fire_streams(0)

      @pl.when(jj > 0)
      def _drain1():
        drain_streams(1)

      fire_in(j0 + 1, 1)
      wait_in(1)
      compute(1)
      fire_streams(1)
      drain_streams(0)

      @pl.when(j0 + 2 < n_sub)
      def _prefetch():
        fire_in(j0 + 2, 0)

    drain_streams(1)

    plsc.subcore_barrier()

    r0 = sid * rows_per_sub
    pltpu.sync_copy(acc.at[pl.ds(r0, rows_per_sub)],
                    out_hbm.at[cid, pl.ds(r0, rows_per_sub)])

  return run


def _combine_kernel(p_ref, b_ref, o_ref):
  s = p_ref[0] + p_ref[1]
  o_ref[...] = s.T + b_ref[...]


@jax.jit
def kernel(x, row_indices, col_indices, values, bias):
  batch, cols_total = x.shape
  rows = bias.shape[0]
  nnz = row_indices.shape[0]
  nnz_pad = math.ceil(nnz / (NUM_WORKERS * 2 * CHUNK)) * NUM_WORKERS * 2 * CHUNK
  pad = nnz_pad - nnz

  cols_p = jnp.concatenate([col_indices, jnp.zeros((pad,), jnp.int32)])
  vals_p = jnp.concatenate([values, jnp.zeros((pad,), jnp.float32)])
  rows_p = jnp.concatenate([row_indices, jnp.zeros((pad,), jnp.int32)])
  rows_2d = rows_p.reshape(nnz_pad // STREAM_ROWS, STREAM_ROWS)
  zeros_tile = jnp.zeros((rows // NUM_SUBCORES, LANES), jnp.float32)

  partials = _sc_accumulate(nnz_pad, rows, cols_total, batch)(
      x.reshape(-1), cols_p, vals_p, rows_2d, zeros_tile)

  out = pl.pallas_call(
      _combine_kernel,
      out_shape=jax.ShapeDtypeStruct((batch, rows), jnp.float32),
  )(partials, bias.reshape(1, rows))
  return out
